# baseline (speedup 1.0000x reference)
import functools

import numpy as np
import jax
import jax.numpy as jnp
from jax import lax
from jax.experimental import pallas as pl
from jax.experimental.pallas import tpu as pltpu
from jax.experimental.pallas import tpu_sc as plsc

_XGRID = np.array([0.0525, 0.07, 0.0875, 0.105, 0.1225, 0.14, 0.1575, 0.175,
                   0.1925, 0.21, 0.2275, 0.245, 0.2625, 0.28, 0.2975, 0.315,
                   0.3325, 0.35], dtype=np.float32)
_YGRID = np.array([644404.0862, 761653.8961, 939789.2253, 1178813.607,
                   1498410.95, 1937406.885, 2546507.201, 3374555.172,
                   4487038.625, 5974375.188, 7940600.158, 10476316.7,
                   13607669.83, 17208745.11, 20945786.46, 24370070.17,
                   27157403.26, 29234740.57], dtype=np.float32)

_XPREV = np.roll(_XGRID, 1)
_YPREV = np.roll(_YGRID, 1)
_SLOPE = (_YGRID - _YPREV) / (_XGRID - _XPREV)
_ICEPT = (_YPREV - _SLOPE * _XPREV).astype(np.float32)

_TBL = 32


def _pad_tbl(a):
    out = np.zeros((_TBL,), dtype=np.float32)
    out[: a.size] = a
    return jnp.asarray(out)


_N = 1_000_000
_NW = 32
_CHUNK = 31_264
_HALF = _CHUNK // 2
_VECS = _CHUNK // 16

_X0P = np.float32(np.float64(0.0525) - np.float64(0.0175) / 2.0)
_INV_H = np.float32(1.0) / np.float32(0.0175)
_DECAY = np.float32(0.1)


def _interp(v, slope_v, icept_v):
    r = (v - _X0P) * _INV_H
    rc = jnp.minimum(jnp.maximum(r, 0.0), np.float32(17.5))
    idx = rc.astype(jnp.int32)
    s = plsc.load_gather(slope_v, [idx])
    c = plsc.load_gather(icept_v, [idx])
    return s * v + c


@functools.cache
def _build_sc_kernel():
    mesh = plsc.VectorSubcoreMesh(core_axis_name="c", subcore_axis_name="s")

    @functools.partial(
        pl.kernel,
        mesh=mesh,
        out_type=jax.ShapeDtypeStruct((_N,), jnp.float32),
        scratch_types=[
            pltpu.VMEM((_CHUNK,), jnp.float32),
            pltpu.VMEM((_CHUNK,), jnp.float32),
            pltpu.VMEM((_CHUNK,), jnp.float32),
            pltpu.VMEM((_CHUNK,), jnp.float32),
            pltpu.VMEM((_TBL,), jnp.float32),
            pltpu.VMEM((_TBL,), jnp.float32),
            pltpu.SemaphoreType.DMA,
            pltpu.SemaphoreType.DMA,
            pltpu.SemaphoreType.DMA,
        ],
        compiler_params=pltpu.CompilerParams(needs_layout_passes=False, skip_device_barrier=True),
    )
    def _sc_kernel(x_hbm, vref_hbm, vcb_hbm, sl_hbm, ic_hbm, out_hbm,
                   x_v, vr_v, vb_v, o_v, sl_v, ic_v, sem0, sem1, sem_o):
        wid = lax.axis_index("s") * 2 + lax.axis_index("c")
        base = jnp.minimum(wid * _CHUNK, _N - _CHUNK)

        sems = (sem0, sem1)
        ins = []
        for k in range(2):
            lo = k * _HALF
            s = pl.ds(base + lo, _HALF)
            d = pl.ds(lo, _HALF)
            ins.append([
                pltpu.async_copy(x_hbm.at[s], x_v.at[d], sems[k]),
                pltpu.async_copy(vref_hbm.at[s], vr_v.at[d], sems[k]),
                pltpu.async_copy(vcb_hbm.at[s], vb_v.at[d], sems[k]),
            ])
        pltpu.sync_copy(sl_hbm, sl_v)
        pltpu.sync_copy(ic_hbm, ic_v)

        out_cp = None
        for k in range(2):
            for c in ins[k]:
                c.wait()
            lo = k * _HALF

            @plsc.parallel_loop(lo, lo + _HALF, 16, unroll=16)
            def body(off):
                sl_ = pl.ds(off, 16)
                xv = x_v[sl_]
                vr = vr_v[sl_]
                vcb = vb_v[sl_]
                vb = jnp.maximum((vcb + xv) - vcb * _DECAY, 0.0)
                f = _interp(vb, sl_v, ic_v)
                l = _interp(vr, sl_v, ic_v)
                p = l * f
                q = (f - l) * (f + l) * np.float32(2.0)
                cond = (q - p) * p >= 0.0
                o_v[sl_] = jnp.where(cond, np.float32(1.0), np.float32(0.0))

            if k == 0:
                out_cp = pltpu.async_copy(
                    o_v.at[pl.ds(0, _HALF)],
                    out_hbm.at[pl.ds(base, _HALF)], sem_o)
        pltpu.sync_copy(o_v.at[pl.ds(_HALF, _HALF)],
                        out_hbm.at[pl.ds(base + _HALF, _HALF)])
        out_cp.wait()

    return _sc_kernel


def kernel(x, vref, vco_bias):
    return _build_sc_kernel()(x, vref, vco_bias,
                              _pad_tbl(_SLOPE), _pad_tbl(_ICEPT))


# speedup vs baseline: 1.5790x; 1.5790x over previous
import functools

import numpy as np
import jax
import jax.numpy as jnp
from jax import lax
from jax.experimental import pallas as pl
from jax.experimental.pallas import tpu as pltpu
from jax.experimental.pallas import tpu_sc as plsc

_XGRID = np.array([0.0525, 0.07, 0.0875, 0.105, 0.1225, 0.14, 0.1575, 0.175,
                   0.1925, 0.21, 0.2275, 0.245, 0.2625, 0.28, 0.2975, 0.315,
                   0.3325, 0.35], dtype=np.float32)
_YGRID = np.array([644404.0862, 761653.8961, 939789.2253, 1178813.607,
                   1498410.95, 1937406.885, 2546507.201, 3374555.172,
                   4487038.625, 5974375.188, 7940600.158, 10476316.7,
                   13607669.83, 17208745.11, 20945786.46, 24370070.17,
                   27157403.26, 29234740.57], dtype=np.float32)

_XPREV = np.roll(_XGRID, 1)
_YPREV = np.roll(_YGRID, 1)
_SLOPE = (_YGRID - _YPREV) / (_XGRID - _XPREV)
_ICEPT = (_YPREV - _SLOPE * _XPREV).astype(np.float32)

_TBL = 32


def _pad_tbl(a):
    out = np.zeros((_TBL,), dtype=np.float32)
    out[: a.size] = a
    return jnp.asarray(out)


_N = 1_000_000
_NW = 32
_CHUNK = 31_264
_HALF = _CHUNK // 2
_VECS = _CHUNK // 16

_X0P = np.float32(np.float64(0.0525) - np.float64(0.0175) / 2.0)
_INV_H = np.float32(1.0) / np.float32(0.0175)
_DECAY = np.float32(0.1)


def _interp(v, slope_v, icept_v):
    r = (v - _X0P) * _INV_H
    rc = jnp.minimum(jnp.maximum(r, 0.0), np.float32(17.5))
    idx = rc.astype(jnp.int32)
    s = plsc.load_gather(slope_v, [idx])
    c = plsc.load_gather(icept_v, [idx])
    return s * v + c


@functools.cache
def _build_sc_kernel():
    mesh = plsc.VectorSubcoreMesh(core_axis_name="c", subcore_axis_name="s")

    @functools.partial(
        pl.kernel,
        mesh=mesh,
        out_type=jax.ShapeDtypeStruct((_N,), jnp.float32),
        scratch_types=[
            pltpu.VMEM((_CHUNK,), jnp.float32),
            pltpu.VMEM((_CHUNK,), jnp.float32),
            pltpu.VMEM((_CHUNK,), jnp.float32),
            pltpu.VMEM((_CHUNK,), jnp.float32),
            pltpu.VMEM((_TBL,), jnp.float32),
            pltpu.VMEM((_TBL,), jnp.float32),
            pltpu.SemaphoreType.DMA,
            pltpu.SemaphoreType.DMA,
            pltpu.SemaphoreType.DMA,
        ],
        compiler_params=pltpu.CompilerParams(needs_layout_passes=False, skip_device_barrier=True),
    )
    def _sc_kernel(x_hbm, vref_hbm, vcb_hbm, sl_hbm, ic_hbm, out_hbm,
                   x_v, vr_v, vb_v, o_v, sl_v, ic_v, sem0, sem1, sem_o):
        wid = lax.axis_index("s") * 2 + lax.axis_index("c")
        base = jnp.minimum(wid * _CHUNK, _N - _CHUNK)

        sems = (sem0, sem1)
        ins = []
        for k in range(2):
            lo = k * _HALF
            s = pl.ds(base + lo, _HALF)
            d = pl.ds(lo, _HALF)
            ins.append([
                pltpu.async_copy(x_hbm.at[s], x_v.at[d], sems[k]),
                pltpu.async_copy(vref_hbm.at[s], vr_v.at[d], sems[k]),
                pltpu.async_copy(vcb_hbm.at[s], vb_v.at[d], sems[k]),
            ])
        pltpu.sync_copy(sl_hbm, sl_v)
        pltpu.sync_copy(ic_hbm, ic_v)

        out_cp = None
        for k in range(2):
            for c in ins[k]:
                c.wait()
            lo = k * _HALF

            @plsc.parallel_loop(lo, lo + _HALF, 16, unroll=4)
            def body(off):
                sl_ = pl.ds(off, 16)
                xv = x_v[sl_]
                vr = vr_v[sl_]
                vcb = vb_v[sl_]
                vb = jnp.maximum((vcb + xv) - vcb * _DECAY, 0.0)
                f = _interp(vb, sl_v, ic_v)
                l = _interp(vr, sl_v, ic_v)
                p = l * f
                q = (f - l) * (f + l) * np.float32(2.0)
                cond = (q - p) * p >= 0.0
                o_v[sl_] = jnp.where(cond, np.float32(1.0), np.float32(0.0))

            if k == 0:
                out_cp = pltpu.async_copy(
                    o_v.at[pl.ds(0, _HALF)],
                    out_hbm.at[pl.ds(base, _HALF)], sem_o)
        pltpu.sync_copy(o_v.at[pl.ds(_HALF, _HALF)],
                        out_hbm.at[pl.ds(base + _HALF, _HALF)])
        out_cp.wait()

    return _sc_kernel


def kernel(x, vref, vco_bias):
    return _build_sc_kernel()(x, vref, vco_bias,
                              _pad_tbl(_SLOPE), _pad_tbl(_ICEPT))
